# baseline (speedup 1.0000x reference)
import functools

import jax
import jax.numpy as jnp
from jax.experimental import pallas as pl
from jax.experimental.pallas import tpu as pltpu


def _elu(x):
    return jnp.where(x > 0, x, jnp.exp(jnp.minimum(x, 0.0)) - 1.0)


def _spmm_body(NB, RL, f_ref, w_ref, adj_ref, b_ref, wp1_ref, bp1_ref,
               wp2_ref, vp1_ref, z_ref, u_ref, w1s_ref, xw_ref, s1acc_ref):
    bf = jnp.bfloat16
    ij = pl.program_id(0)
    n = pl.program_id(1)
    slot = ij % 2

    @pl.when((ij == 0) & (n == 0))
    def _():
        xw = jnp.dot(f_ref[...].astype(bf), w_ref[0].astype(bf),
                     preferred_element_type=jnp.float32)
        xw_ref[0] = xw.astype(bf)

    a16 = adj_ref[0].astype(bf)
    acc = jnp.dot(a16, xw_ref[slot], preferred_element_type=jnp.float32)

    @pl.when((n == NB - 1) & (ij < RL - 1))
    def _():
        xw_nxt = jnp.dot(f_ref[...].astype(bf), w_ref[0].astype(bf),
                         preferred_element_type=jnp.float32)
        xw_ref[1 - slot] = xw_nxt.astype(bf)
    z = _elu(acc + b_ref[0])
    z16 = z.astype(bf)
    z_ref[0] = z16
    t = jnp.tanh(jnp.dot(z16, wp1_ref[0].astype(bf),
                         preferred_element_type=jnp.float32)
                 + bp1_ref[0])
    s1_blk = jnp.dot(t.astype(bf), wp2_ref[0].astype(bf),
                     preferred_element_type=jnp.float32)
    u_ref[0] = jnp.dot(z16, vp1_ref[...].astype(bf),
                       preferred_element_type=jnp.float32).astype(bf)

    part = jnp.sum(s1_blk, axis=(0, 1), keepdims=True)
    prev = jnp.where(n == 0, jnp.zeros((1, 1), jnp.float32), s1acc_ref[...])
    tot = prev + part
    s1acc_ref[...] = tot

    @pl.when(n == NB - 1)
    def _():
        w1s_ref[0] = tot


def _beta1_from_sums(w1s, R, L, N):
    w1m = w1s.reshape(R, L) / N
    w1m = w1m - jnp.max(w1m, axis=1, keepdims=True)
    e = jnp.exp(w1m)
    return e / jnp.sum(e, axis=1, keepdims=True)


def _cd_body(R, L, N, NC, u_ref, z_ref, w1s_ref, vb1_ref, vp2_ref,
             out_ref, w2acc_ref):
    bf = jnp.bfloat16
    n = pl.program_id(0)
    beta1 = _beta1_from_sums(w1s_ref[...], R, L, N)

    @pl.when(n < NC)
    def _():
        for i in range(R):
            hv = beta1[i, 0] * u_ref[i * L].astype(jnp.float32)
            for j in range(1, L):
                hv = hv + beta1[i, j] * u_ref[i * L + j].astype(jnp.float32)
            t = jnp.tanh(hv + vb1_ref[...])
            s2_blk = jnp.dot(t.astype(bf), vp2_ref[...].astype(bf),
                             preferred_element_type=jnp.float32)
            part = jnp.sum(s2_blk, axis=(0, 1), keepdims=True)
            prev = jnp.where(n == 0, jnp.zeros((1, 1), jnp.float32),
                             w2acc_ref[:, i:i + 1])
            w2acc_ref[:, i:i + 1] = prev + part

    @pl.when(n >= NC)
    def _():
        w2m = w2acc_ref[...] / N
        w2m = w2m - jnp.max(w2m)
        e2 = jnp.exp(w2m)
        beta2 = (e2 / jnp.sum(e2)).reshape(R, 1)
        c = (beta2 * beta1).reshape(R * L)
        acc = c[0] * z_ref[0].astype(jnp.float32)
        for k in range(1, R * L):
            acc = acc + c[k] * z_ref[k].astype(jnp.float32)
        out_ref[...] = acc


def kernel(features, ADJ, W, b, Wp1, bp1, Wp2, Vp1, vb1, Vp2):
    R, L, N, _ = ADJ.shape
    D = features.shape[1]
    H = Wp1.shape[2]
    RL = R * L
    BN = min(1024, N)
    NB = N // BN
    BC = min(4096, N)
    NC = N // BC

    bf = jnp.bfloat16
    ADJ3 = ADJ.reshape(RL, N, N)
    W3 = W.reshape(RL, D, D)
    b2 = b.reshape(RL, 1, D)
    bp1_3 = bp1.reshape(R, 1, H)
    vb1_2 = vb1.reshape(1, H)

    z, u, w1s = pl.pallas_call(
        functools.partial(_spmm_body, NB, RL),
        grid=(RL, NB),
        in_specs=[
            pl.BlockSpec((N, D), lambda ij, n: (0, 0)),
            pl.BlockSpec(
                (1, D, D),
                lambda ij, n: (jnp.where(n == NB - 1,
                                         jnp.minimum(ij + 1, RL - 1), ij),
                               0, 0)),
            pl.BlockSpec((1, BN, N), lambda ij, n: (ij, n, 0)),
            pl.BlockSpec((1, 1, D), lambda ij, n: (ij, 0, 0)),
            pl.BlockSpec((1, D, H), lambda ij, n: (ij // L, 0, 0)),
            pl.BlockSpec((1, 1, H), lambda ij, n: (ij // L, 0, 0)),
            pl.BlockSpec((1, H, 1), lambda ij, n: (ij // L, 0, 0)),
            pl.BlockSpec((D, H), lambda ij, n: (0, 0)),
        ],
        out_specs=[
            pl.BlockSpec((1, BN, D), lambda ij, n: (ij, n, 0)),
            pl.BlockSpec((1, BN, H), lambda ij, n: (ij, n, 0)),
            pl.BlockSpec((1, 1, 1), lambda ij, n: (ij, 0, 0)),
        ],
        out_shape=[
            jax.ShapeDtypeStruct((RL, N, D), bf),
            jax.ShapeDtypeStruct((RL, N, H), bf),
            jax.ShapeDtypeStruct((RL, 1, 1), jnp.float32),
        ],
        scratch_shapes=[pltpu.VMEM((2, N, D), bf),
                        pltpu.VMEM((1, 1), jnp.float32)],
        compiler_params=pltpu.CompilerParams(
            vmem_limit_bytes=66_000_000),
    )(features, W3, ADJ3, b2, Wp1, bp1_3, Wp2, Vp1)

    out = pl.pallas_call(
        functools.partial(_cd_body, R, L, N, NC),
        grid=(2 * NC,),
        in_specs=[
            pl.BlockSpec((RL, BC, H), lambda n: (0, jnp.minimum(n, NC - 1), 0)),
            pl.BlockSpec((RL, BC, D), lambda n: (0, jnp.maximum(n - NC, 0), 0)),
            pl.BlockSpec((RL, 1, 1), lambda n: (0, 0, 0)),
            pl.BlockSpec((1, H), lambda n: (0, 0)),
            pl.BlockSpec((H, 1), lambda n: (0, 0)),
        ],
        out_specs=pl.BlockSpec((BC, D), lambda n: (jnp.maximum(n - NC, 0), 0)),
        out_shape=jax.ShapeDtypeStruct((N, D), jnp.float32),
        scratch_shapes=[pltpu.VMEM((1, R), jnp.float32)],
    )(u, z, w1s, vb1_2, Vp2)

    return out


# speedup vs baseline: 1.0725x; 1.0725x over previous
import functools

import jax
import jax.numpy as jnp
from jax.experimental import pallas as pl
from jax.experimental.pallas import tpu as pltpu


def _elu(x):
    return jnp.where(x > 0, x, jnp.exp(jnp.minimum(x, 0.0)) - 1.0)


def _spmm_body(NB, f_ref, w_ref, adj_ref, b_ref, wp1_ref, bp1_ref, wp2_ref,
               vp1_ref, z_ref, u_ref, w1s_ref, xw_ref, s1acc_ref):
    bf = jnp.bfloat16
    n = pl.program_id(1)

    @pl.when(n == 0)
    def _():
        xw = jnp.dot(f_ref[...].astype(bf), w_ref[0].astype(bf),
                     preferred_element_type=jnp.float32)
        xw_ref[...] = xw.astype(bf)

    a16 = adj_ref[0].astype(bf)
    acc = jnp.dot(a16, xw_ref[...], preferred_element_type=jnp.float32)
    z = _elu(acc + b_ref[0])
    z16 = z.astype(bf)
    z_ref[0] = z16
    t = jnp.tanh(jnp.dot(z16, wp1_ref[0].astype(bf),
                         preferred_element_type=jnp.float32)
                 + bp1_ref[0])
    s1_blk = jnp.dot(t.astype(bf), wp2_ref[0].astype(bf),
                     preferred_element_type=jnp.float32)
    u_ref[0] = jnp.dot(z16, vp1_ref[...].astype(bf),
                       preferred_element_type=jnp.float32).astype(bf)

    part = jnp.sum(s1_blk, axis=(0, 1), keepdims=True)
    prev = jnp.where(n == 0, jnp.zeros((1, 1), jnp.float32), s1acc_ref[...])
    tot = prev + part
    s1acc_ref[...] = tot

    @pl.when(n == NB - 1)
    def _():
        w1s_ref[0] = tot


def _beta1_from_sums(w1s, R, L, N):
    w1m = w1s.reshape(R, L) / N
    w1m = w1m - jnp.max(w1m, axis=1, keepdims=True)
    e = jnp.exp(w1m)
    return e / jnp.sum(e, axis=1, keepdims=True)


def _cd_body(R, L, N, NC, u_ref, z_ref, w1s_ref, vb1_ref, vp2_ref,
             out_ref, w2acc_ref):
    bf = jnp.bfloat16
    n = pl.program_id(0)
    beta1 = _beta1_from_sums(w1s_ref[...], R, L, N)

    @pl.when(n < NC)
    def _():
        for i in range(R):
            hv = beta1[i, 0] * u_ref[i * L].astype(jnp.float32)
            for j in range(1, L):
                hv = hv + beta1[i, j] * u_ref[i * L + j].astype(jnp.float32)
            t = jnp.tanh(hv + vb1_ref[...])
            s2_blk = jnp.dot(t.astype(bf), vp2_ref[...].astype(bf),
                             preferred_element_type=jnp.float32)
            part = jnp.sum(s2_blk, axis=(0, 1), keepdims=True)
            prev = jnp.where(n == 0, jnp.zeros((1, 1), jnp.float32),
                             w2acc_ref[:, i:i + 1])
            w2acc_ref[:, i:i + 1] = prev + part

    @pl.when(n >= NC)
    def _():
        w2m = w2acc_ref[...] / N
        w2m = w2m - jnp.max(w2m)
        e2 = jnp.exp(w2m)
        beta2 = (e2 / jnp.sum(e2)).reshape(R, 1)
        c = (beta2 * beta1).reshape(R * L)
        acc = c[0] * z_ref[0].astype(jnp.float32)
        for k in range(1, R * L):
            acc = acc + c[k] * z_ref[k].astype(jnp.float32)
        out_ref[...] = acc


def kernel(features, ADJ, W, b, Wp1, bp1, Wp2, Vp1, vb1, Vp2):
    R, L, N, _ = ADJ.shape
    D = features.shape[1]
    H = Wp1.shape[2]
    RL = R * L
    BN = min(1024, N)
    NB = N // BN
    BC = min(4096, N)
    NC = N // BC

    bf = jnp.bfloat16
    ADJ3 = ADJ.reshape(RL, N, N)
    W3 = W.reshape(RL, D, D)
    b2 = b.reshape(RL, 1, D)
    bp1_3 = bp1.reshape(R, 1, H)
    vb1_2 = vb1.reshape(1, H)

    z, u, w1s = pl.pallas_call(
        functools.partial(_spmm_body, NB),
        grid=(RL, NB),
        in_specs=[
            pl.BlockSpec((N, D), lambda ij, n: (0, 0)),
            pl.BlockSpec((1, D, D), lambda ij, n: (ij, 0, 0)),
            pl.BlockSpec((1, BN, N), lambda ij, n: (ij, n, 0)),
            pl.BlockSpec((1, 1, D), lambda ij, n: (ij, 0, 0)),
            pl.BlockSpec((1, D, H), lambda ij, n: (ij // L, 0, 0)),
            pl.BlockSpec((1, 1, H), lambda ij, n: (ij // L, 0, 0)),
            pl.BlockSpec((1, H, 1), lambda ij, n: (ij // L, 0, 0)),
            pl.BlockSpec((D, H), lambda ij, n: (0, 0)),
        ],
        out_specs=[
            pl.BlockSpec((1, BN, D), lambda ij, n: (ij, n, 0)),
            pl.BlockSpec((1, BN, H), lambda ij, n: (ij, n, 0)),
            pl.BlockSpec((1, 1, 1), lambda ij, n: (ij, 0, 0)),
        ],
        out_shape=[
            jax.ShapeDtypeStruct((RL, N, D), bf),
            jax.ShapeDtypeStruct((RL, N, H), bf),
            jax.ShapeDtypeStruct((RL, 1, 1), jnp.float32),
        ],
        scratch_shapes=[pltpu.VMEM((N, D), bf),
                        pltpu.VMEM((1, 1), jnp.float32)],
    )(features, W3, ADJ3, b2, Wp1, bp1_3, Wp2, Vp1)

    out = pl.pallas_call(
        functools.partial(_cd_body, R, L, N, NC),
        grid=(2 * NC,),
        in_specs=[
            pl.BlockSpec((RL, BC, H), lambda n: (0, jnp.minimum(n, NC - 1), 0)),
            pl.BlockSpec((RL, BC, D), lambda n: (0, jnp.maximum(n - NC, 0), 0)),
            pl.BlockSpec((RL, 1, 1), lambda n: (0, 0, 0)),
            pl.BlockSpec((1, H), lambda n: (0, 0)),
            pl.BlockSpec((H, 1), lambda n: (0, 0)),
        ],
        out_specs=pl.BlockSpec((BC, D), lambda n: (jnp.maximum(n - NC, 0), 0)),
        out_shape=jax.ShapeDtypeStruct((N, D), jnp.float32),
        scratch_shapes=[pltpu.VMEM((1, R), jnp.float32)],
    )(u, z, w1s, vb1_2, Vp2)

    return out


# speedup vs baseline: 1.0850x; 1.0116x over previous
import functools

import jax
import jax.numpy as jnp
from jax.experimental import pallas as pl
from jax.experimental.pallas import tpu as pltpu


def _elu(x):
    return jnp.where(x > 0, x, jnp.exp(jnp.minimum(x, 0.0)) - 1.0)


def _spmm_body(NB, f_ref, w_ref, adj_ref, b_ref, wp1_ref, bp1_ref, wp2_ref,
               z_ref, w1s_ref, xw_ref, s1acc_ref):
    bf = jnp.bfloat16
    n = pl.program_id(1)

    @pl.when(n == 0)
    def _():
        xw = jnp.dot(f_ref[...].astype(bf), w_ref[0].astype(bf),
                     preferred_element_type=jnp.float32)
        xw_ref[...] = xw.astype(bf)

    a16 = adj_ref[0].astype(bf)
    acc = jnp.dot(a16, xw_ref[...], preferred_element_type=jnp.float32)
    z = _elu(acc + b_ref[0])
    z16 = z.astype(bf)
    z_ref[0] = z16
    t = jnp.tanh(jnp.dot(z16, wp1_ref[0].astype(bf),
                         preferred_element_type=jnp.float32)
                 + bp1_ref[0])
    s1_blk = jnp.dot(t.astype(bf), wp2_ref[0].astype(bf),
                     preferred_element_type=jnp.float32)

    part = jnp.sum(s1_blk, axis=(0, 1), keepdims=True)
    prev = jnp.where(n == 0, jnp.zeros((1, 1), jnp.float32), s1acc_ref[...])
    tot = prev + part
    s1acc_ref[...] = tot

    @pl.when(n == NB - 1)
    def _():
        w1s_ref[0] = tot


def _beta1_from_sums(w1s, R, L, N):
    w1m = w1s.reshape(R, L) / N
    w1m = w1m - jnp.max(w1m, axis=1, keepdims=True)
    e = jnp.exp(w1m)
    return e / jnp.sum(e, axis=1, keepdims=True)


def _cd_body(R, L, N, z_ref, w1s_ref, vp1_ref, vb1_ref, vp2_ref,
             out_ref, w2acc_ref):
    bf = jnp.bfloat16
    n = pl.program_id(0)
    beta1 = _beta1_from_sums(w1s_ref[...], R, L, N)

    @pl.when(n == 0)
    def _():
        for i in range(R):
            h = beta1[i, 0] * z_ref[i * L].astype(jnp.float32)
            for j in range(1, L):
                h = h + beta1[i, j] * z_ref[i * L + j].astype(jnp.float32)
            t = jnp.tanh(jnp.dot(h.astype(bf), vp1_ref[...].astype(bf),
                                 preferred_element_type=jnp.float32)
                         + vb1_ref[...])
            s2_blk = jnp.dot(t.astype(bf), vp2_ref[...].astype(bf),
                             preferred_element_type=jnp.float32)
            w2acc_ref[:, i:i + 1] = jnp.sum(s2_blk, axis=(0, 1), keepdims=True)

    @pl.when(n == 1)
    def _():
        w2m = w2acc_ref[...] / N
        w2m = w2m - jnp.max(w2m)
        e2 = jnp.exp(w2m)
        beta2 = (e2 / jnp.sum(e2)).reshape(R, 1)
        c = (beta2 * beta1).reshape(R * L)
        acc = c[0] * z_ref[0].astype(jnp.float32)
        for k in range(1, R * L):
            acc = acc + c[k] * z_ref[k].astype(jnp.float32)
        out_ref[...] = acc


def kernel(features, ADJ, W, b, Wp1, bp1, Wp2, Vp1, vb1, Vp2):
    R, L, N, _ = ADJ.shape
    D = features.shape[1]
    H = Wp1.shape[2]
    RL = R * L
    BN = min(1024, N)
    NB = N // BN

    bf = jnp.bfloat16
    ADJ3 = ADJ.reshape(RL, N, N)
    W3 = W.reshape(RL, D, D)
    b2 = b.reshape(RL, 1, D)
    bp1_3 = bp1.reshape(R, 1, H)
    vb1_2 = vb1.reshape(1, H)

    z, w1s = pl.pallas_call(
        functools.partial(_spmm_body, NB),
        grid=(RL, NB),
        in_specs=[
            pl.BlockSpec((N, D), lambda ij, n: (0, 0)),
            pl.BlockSpec((1, D, D), lambda ij, n: (ij, 0, 0)),
            pl.BlockSpec((1, BN, N), lambda ij, n: (ij, n, 0)),
            pl.BlockSpec((1, 1, D), lambda ij, n: (ij, 0, 0)),
            pl.BlockSpec((1, D, H), lambda ij, n: (ij // L, 0, 0)),
            pl.BlockSpec((1, 1, H), lambda ij, n: (ij // L, 0, 0)),
            pl.BlockSpec((1, H, 1), lambda ij, n: (ij // L, 0, 0)),
        ],
        out_specs=[
            pl.BlockSpec((1, BN, D), lambda ij, n: (ij, n, 0)),
            pl.BlockSpec((1, 1, 1), lambda ij, n: (ij, 0, 0)),
        ],
        out_shape=[
            jax.ShapeDtypeStruct((RL, N, D), bf),
            jax.ShapeDtypeStruct((RL, 1, 1), jnp.float32),
        ],
        scratch_shapes=[pltpu.VMEM((N, D), bf),
                        pltpu.VMEM((1, 1), jnp.float32)],
    )(features, W3, ADJ3, b2, Wp1, bp1_3, Wp2)

    out = pl.pallas_call(
        functools.partial(_cd_body, R, L, N),
        grid=(2,),
        in_specs=[
            pl.BlockSpec((RL, N, D), lambda n: (0, 0, 0)),
            pl.BlockSpec((RL, 1, 1), lambda n: (0, 0, 0)),
            pl.BlockSpec((D, H), lambda n: (0, 0)),
            pl.BlockSpec((1, H), lambda n: (0, 0)),
            pl.BlockSpec((H, 1), lambda n: (0, 0)),
        ],
        out_specs=pl.BlockSpec((N, D), lambda n: (0, 0)),
        out_shape=jax.ShapeDtypeStruct((N, D), jnp.float32),
        scratch_shapes=[pltpu.VMEM((1, R), jnp.float32)],
    )(z, w1s, Vp1, vb1_2, Vp2)

    return out
